# baseline (speedup 1.0000x reference)
import functools
import jax
import jax.numpy as jnp
from jax import lax
from jax.experimental import pallas as pl
from jax.experimental.pallas import tpu as pltpu, tpu_sc as plsc

NC = 2
NS = 16
NW = NC * NS

B = 4096
L = 50
H = 32
TOTAL = B * L
B_PER_W = TOTAL // NW
CHUNK = 128
NCHUNK = B_PER_W // CHUNK

_mesh = plsc.VectorSubcoreMesh(core_axis_name="c", subcore_axis_name="s")


@functools.partial(
    pl.kernel,
    out_type=jax.ShapeDtypeStruct((TOTAL, H), jnp.float32),
    mesh=_mesh,
    scratch_types=[
        pltpu.VMEM((NCHUNK, CHUNK), jnp.int32),
        pltpu.VMEM((CHUNK, H), jnp.float32),
        pltpu.SemaphoreType.DMA,
    ],
    compiler_params=pltpu.CompilerParams(use_tc_tiling_on_sc=False),
)
def _gather_kernel(idx_hbm, table_hbm, out_hbm, idx_v, rows_v, sem):
    wid = lax.axis_index("s") * NC + lax.axis_index("c")
    base = wid * B_PER_W
    pltpu.sync_copy(idx_hbm.at[wid], idx_v)

    def body(j, carry):
        pltpu.async_copy(table_hbm.at[idx_v.at[j]], rows_v, sem).wait()
        pltpu.sync_copy(rows_v, out_hbm.at[pl.ds(base + j * CHUNK, CHUNK)])
        return carry

    lax.fori_loop(0, NCHUNK, body, 0)


def kernel(conditions, table):
    idx = conditions.reshape(NW, NCHUNK, CHUNK)
    out = _gather_kernel(idx, table)
    return out.reshape(B, L * H)


# speedup vs baseline: 1.0621x; 1.0621x over previous
import functools
import jax
import jax.numpy as jnp
from jax import lax
from jax.experimental import pallas as pl
from jax.experimental.pallas import tpu as pltpu, tpu_sc as plsc

NC = 2
NS = 16
NW = NC * NS

B = 4096
L = 50
H = 32
TOTAL = B * L
B_PER_W = TOTAL // NW
CHUNK = 128
NCHUNK = B_PER_W // CHUNK

NB = 8
LOOKAHEAD = 4

_mesh = plsc.VectorSubcoreMesh(core_axis_name="c", subcore_axis_name="s")


@functools.partial(
    pl.kernel,
    out_type=jax.ShapeDtypeStruct((TOTAL, H), jnp.float32),
    mesh=_mesh,
    scratch_types=[
        pltpu.VMEM((NCHUNK, CHUNK), jnp.int32),
        pltpu.VMEM((NB, CHUNK, H), jnp.float32),
        pltpu.SemaphoreType.DMA((NB,)),
        pltpu.SemaphoreType.DMA((NB,)),
    ],
    compiler_params=pltpu.CompilerParams(use_tc_tiling_on_sc=False),
)
def _gather_kernel(idx_hbm, table_hbm, out_hbm, idx_v, rows_v, gsems, osems):
    wid = lax.axis_index("s") * NC + lax.axis_index("c")
    base = wid * B_PER_W
    pltpu.sync_copy(idx_hbm.at[wid], idx_v)

    def gather_start(j, b):
        pltpu.async_copy(table_hbm.at[idx_v.at[j]], rows_v.at[b], gsems.at[b])

    def gather_wait(j, b):
        pltpu.make_async_copy(
            table_hbm.at[idx_v.at[j]], rows_v.at[b], gsems.at[b]
        ).wait()

    def out_start(j, b):
        pltpu.async_copy(
            rows_v.at[b], out_hbm.at[pl.ds(base + j * CHUNK, CHUNK)], osems.at[b]
        )

    def out_wait(j, b):
        pltpu.make_async_copy(
            rows_v.at[b], out_hbm.at[pl.ds(base + j * CHUNK, CHUNK)], osems.at[b]
        ).wait()

    for b in range(LOOKAHEAD):
        gather_start(b, b)

    def body(j, carry):
        b = j % NB
        gather_wait(j, b)
        out_start(j, b)
        jn = j + LOOKAHEAD
        bn = jn % NB

        @pl.when(jn < NCHUNK)
        def _():
            @pl.when(jn >= NB)
            def _():
                out_wait(jn - NB, bn)

            gather_start(jn, bn)

        return carry

    lax.fori_loop(0, NCHUNK, body, 0)

    for t in range(NCHUNK - NB, NCHUNK):
        out_wait(t, t % NB)


def kernel(conditions, table):
    idx = conditions.reshape(NW, NCHUNK, CHUNK)
    out = _gather_kernel(idx, table)
    return out.reshape(B, L * H)
